# baseline (speedup 1.0000x reference)
import functools

import jax
import jax.numpy as jnp
from jax import lax
from jax.experimental import pallas as pl
from jax.experimental.pallas import tpu as pltpu
from jax.experimental.pallas import tpu_sc as plsc

K_Q = 65536
D = 128
B_K = 4096
MASK = K_Q - 1
NS = 8
PC = B_K // NS
IL = 128


def _sc_body(keys_hbm, ptr_hbm, out_hbm, ptr_v, idx_r, buf_v, sem):
    w = lax.axis_index("s")
    tp = pl.multiple_of(w * PC, PC)
    pltpu.sync_copy(ptr_hbm, ptr_v.at[pl.ds(0, 1)])
    p = ptr_v[...][0]
    dp = (p + tp) & MASK
    lin = jnp.logical_and((p & 7) == 0, dp <= K_Q - PC)

    pltpu.sync_copy(keys_hbm.at[pl.ds(tp, PC)], buf_v)

    @pl.when(lin)
    def _():
        dpa = pl.multiple_of(dp, 8)
        pltpu.sync_copy(buf_v, out_hbm.at[pl.ds(dpa, PC)])

    @pl.when(jnp.logical_not(lin))
    def _():
        for h in range(PC // IL):
            base = dp + h * IL
            for q in range(IL // 16):
                idx_r[pl.ds(q * 16, 16)] = (
                    base + q * 16 + lax.iota(jnp.int32, 16)) & MASK
            pltpu.async_copy(buf_v.at[pl.ds(h * IL, IL)],
                             out_hbm.at[idx_r], sem).wait()


def kernel(keys, queue, ptr):
    mesh = plsc.VectorSubcoreMesh(
        core_axis_name="c", subcore_axis_name="s",
        num_cores=1, num_subcores=NS)
    enqueue = functools.partial(
        pl.kernel,
        mesh=mesh,
        scratch_types=[
            pltpu.VMEM((16,), jnp.int32),
            pltpu.VMEM((IL,), jnp.int32),
            pltpu.VMEM((PC, D), jnp.float32),
            pltpu.SemaphoreType.DMA,
        ],
    )(_sc_body)
    out_ref = jax.new_ref(queue)
    enqueue(keys, ptr.astype(jnp.int32), out_ref)
    return out_ref[...]


# speedup vs baseline: 1.0668x; 1.0668x over previous
import functools

import jax
import jax.numpy as jnp
from jax import lax
from jax.experimental import pallas as pl
from jax.experimental.pallas import tpu as pltpu
from jax.experimental.pallas import tpu_sc as plsc

K_Q = 65536
D = 128
B_K = 4096
MASK = K_Q - 1
NS = 16
PC = B_K // NS
IL = 128


def _sc_body(keys_hbm, ptr_hbm, out_hbm, ptr_v, idx_a, idx_b, buf_v, sem):
    w = lax.axis_index("s")
    tp = pl.multiple_of(w * PC, PC)
    pltpu.sync_copy(ptr_hbm, ptr_v.at[pl.ds(0, 1)])
    p = ptr_v[...][0]
    dp = (p + tp) & MASK
    lin = jnp.logical_and((p & 7) == 0, dp <= K_Q - PC)

    pltpu.sync_copy(keys_hbm.at[pl.ds(tp, PC)], buf_v)

    @pl.when(lin)
    def _():
        dpa = pl.multiple_of(dp, 8)
        pltpu.sync_copy(buf_v, out_hbm.at[pl.ds(dpa, PC)])

    @pl.when(jnp.logical_not(lin))
    def _():
        for h, idx_r in enumerate((idx_a, idx_b)):
            base = dp + h * IL
            for q in range(IL // 16):
                idx_r[pl.ds(q * 16, 16)] = (
                    base + q * 16 + lax.iota(jnp.int32, 16)) & MASK
            pltpu.async_copy(buf_v.at[pl.ds(h * IL, IL)],
                             out_hbm.at[idx_r], sem).wait()


def kernel(keys, queue, ptr):
    mesh = plsc.VectorSubcoreMesh(
        core_axis_name="c", subcore_axis_name="s", num_cores=1)
    enqueue = functools.partial(
        pl.kernel,
        mesh=mesh,
        scratch_types=[
            pltpu.VMEM((16,), jnp.int32),
            pltpu.VMEM((IL,), jnp.int32),
            pltpu.VMEM((IL,), jnp.int32),
            pltpu.VMEM((PC, D), jnp.float32),
            pltpu.SemaphoreType.DMA,
        ],
    )(_sc_body)
    out_ref = jax.new_ref(queue)
    enqueue(keys, ptr.astype(jnp.int32), out_ref)
    return out_ref[...]
